# baseline (speedup 1.0000x reference)
import functools

import jax
import jax.numpy as jnp
from jax import lax
from jax.experimental import pallas as pl
from jax.experimental.pallas import tpu as pltpu
from jax.experimental.pallas import tpu_sc as plsc

N_NODES = 100000
N_EDGES = 3200000
D_EDGE = 16
D_NODE = 128
D_GLOBAL = 32

_BT = 25600


def _prep_body(ea_ref, *col_refs):
    xt = ea_ref[...].T
    for t in range(D_EDGE):
        col_refs[t][...] = xt[t, :]


def _prep(edge_attributes):
    nb = N_EDGES // _BT
    return pl.pallas_call(
        _prep_body,
        grid=(nb,),
        in_specs=[pl.BlockSpec((_BT, D_EDGE), lambda i: (i, 0))],
        out_specs=[pl.BlockSpec((_BT,), lambda i: (i,))] * D_EDGE,
        out_shape=[jax.ShapeDtypeStruct((N_EDGES,), jnp.float32)] * D_EDGE,
    )(edge_attributes)


_CK = 3200
_NCH = N_EDGES // _CK
_G = _CK // 16


_NBUF = 4


def _seg2_body(ei_hbm, *rest):
    col_hbms = rest[:D_EDGE]
    rect_hbm, sent_hbm = rest[D_EDGE], rest[D_EDGE + 1]
    scratch = rest[D_EDGE + 2:]
    idx_vs = scratch[:_NBUF]
    col_vs = scratch[_NBUF:2 * _NBUF]
    (acc, semi, semc) = scratch[2 * _NBUF:]

    c = lax.axis_index("c")
    s = lax.axis_index("s")

    z = jnp.zeros((16,), jnp.float32)

    @pl.loop(0, N_NODES // 16)
    def _(i):
        acc[pl.ds(i * 16, 16)] = z

    def issue(k, idx_v, col_v):
        @pl.when(c == 0)
        def _():
            pltpu.async_copy(ei_hbm.at[1, pl.ds(k * _CK, _CK)], idx_v, semi)

        @pl.when(c == 1)
        def _():
            pltpu.async_copy(ei_hbm.at[0, pl.ds(k * _CK, _CK)], idx_v, semi)

        for t in range(D_EDGE):
            @pl.when(s == t)
            def _():
                pltpu.async_copy(col_hbms[t].at[pl.ds(k * _CK, _CK)], col_v, semc)

    def wait(idx_v, col_v):
        pltpu.make_async_copy(ei_hbm.at[0, pl.ds(0, _CK)], idx_v, semi).wait()
        pltpu.make_async_copy(col_hbms[0].at[pl.ds(0, _CK)], col_v, semc).wait()

    def compute(idx_v, col_v):
        @plsc.parallel_loop(0, _G, unroll=8)
        def _(g):
            idx = idx_v[pl.ds(g * 16, 16)]
            v = col_v[pl.ds(g * 16, 16)]
            plsc.addupdate_scatter(acc, [idx], v)

    for b in range(_NBUF):
        issue(b, idx_vs[b], col_vs[b])

    @pl.loop(0, _NCH, step=_NBUF)
    def _(k):
        for b in range(_NBUF):
            wait(idx_vs[b], col_vs[b])
            compute(idx_vs[b], col_vs[b])

            @pl.when(k + b + _NBUF < _NCH)
            def _():
                issue(k + b + _NBUF, idx_vs[b], col_vs[b])

    @pl.when(c == 0)
    def _():
        pltpu.sync_copy(acc, rect_hbm.at[s])

    @pl.when(c == 1)
    def _():
        pltpu.sync_copy(acc, sent_hbm.at[s])


_seg2 = functools.partial(
    pl.kernel,
    out_type=[jax.ShapeDtypeStruct((D_EDGE, N_NODES), jnp.float32),
              jax.ShapeDtypeStruct((D_EDGE, N_NODES), jnp.float32)],
    mesh=plsc.VectorSubcoreMesh(core_axis_name="c", subcore_axis_name="s"),
    scratch_types=[pltpu.VMEM((_CK,), jnp.int32)] * _NBUF
    + [pltpu.VMEM((_CK,), jnp.float32)] * _NBUF
    + [
        pltpu.VMEM((N_NODES,), jnp.float32),
        pltpu.SemaphoreType.DMA,
        pltpu.SemaphoreType.DMA,
    ],
    compiler_params=pltpu.CompilerParams(use_tc_tiling_on_sc=False,
                                         needs_layout_passes=False),
)(_seg2_body)


_BM = 2048


def _mm_body(rect_ref, sent_ref, node_ref, g_ref, w_ref, b_ref, out_ref):
    acc = jnp.dot(node_ref[...], w_ref[2 * D_EDGE:2 * D_EDGE + D_NODE, :],
                  preferred_element_type=jnp.float32)
    acc += lax.dot_general(rect_ref[...], w_ref[:D_EDGE, :],
                           (((0,), (0,)), ((), ())),
                           preferred_element_type=jnp.float32)
    acc += lax.dot_general(sent_ref[...], w_ref[D_EDGE:2 * D_EDGE, :],
                           (((0,), (0,)), ((), ())),
                           preferred_element_type=jnp.float32)
    acc += jnp.dot(g_ref[...], w_ref[2 * D_EDGE + D_NODE:, :],
                   preferred_element_type=jnp.float32)
    out_ref[...] = acc + b_ref[...]


def _matmul(rect, sent, node, g2, w, b2):
    d_in = 2 * D_EDGE + D_NODE + D_GLOBAL
    return pl.pallas_call(
        _mm_body,
        grid=(pl.cdiv(N_NODES, _BM),),
        in_specs=[
            pl.BlockSpec((D_EDGE, _BM), lambda i: (0, i)),
            pl.BlockSpec((D_EDGE, _BM), lambda i: (0, i)),
            pl.BlockSpec((_BM, D_NODE), lambda i: (i, 0)),
            pl.BlockSpec((1, D_GLOBAL), lambda i: (0, 0)),
            pl.BlockSpec((d_in, D_NODE), lambda i: (0, 0)),
            pl.BlockSpec((1, D_NODE), lambda i: (0, 0)),
        ],
        out_specs=pl.BlockSpec((_BM, D_NODE), lambda i: (i, 0)),
        out_shape=jax.ShapeDtypeStruct((N_NODES, D_NODE), jnp.float32),
    )(rect, sent, node, g2, w, b2)


def kernel(node_attributes, edge_attributes, global_attributes, edge_index, W, b):
    cols = _prep(edge_attributes)
    rect, sent = _seg2(edge_index, *cols)
    return _matmul(rect, sent, node_attributes,
                   global_attributes.reshape(1, D_GLOBAL), W,
                   b.reshape(1, D_NODE))


# speedup vs baseline: 3.0061x; 3.0061x over previous
import functools

import jax
import jax.numpy as jnp
from jax import lax
from jax.experimental import pallas as pl
from jax.experimental.pallas import tpu as pltpu
from jax.experimental.pallas import tpu_sc as plsc

N_NODES = 100000
N_EDGES = 3200000
D_EDGE = 16
D_NODE = 128
D_GLOBAL = 32

_RB = 25
_CK = _RB * 128
_NCH = N_EDGES // _CK
_NRB = N_EDGES // 128
_NBUF = 4


def _seg_body(ei_hbm, ea_hbm, rect_hbm, sent_hbm, *scratch):
    idx_vs = scratch[:_NBUF]
    col_vs = scratch[_NBUF:2 * _NBUF]
    (acc, semi, semc) = scratch[2 * _NBUF:]

    c = lax.axis_index("c")
    s = lax.axis_index("s")
    th = s // 8
    tr = s % 8

    z = jnp.zeros((16,), jnp.float32)

    @pl.loop(0, N_NODES // 16)
    def _(i):
        acc[pl.ds(i * 16, 16)] = z

    def issue(k, idx_v, col_v):
        @pl.when(c == 0)
        def _():
            pltpu.async_copy(ei_hbm.at[pl.ds(k * _RB, _RB), 1], idx_v, semi)

        @pl.when(c == 1)
        def _():
            pltpu.async_copy(ei_hbm.at[pl.ds(k * _RB, _RB), 0], idx_v, semi)

        pltpu.async_copy(ea_hbm.at[th, pl.ds(k * _RB, _RB), tr], col_v, semc)

    def wait(idx_v, col_v):
        pltpu.make_async_copy(ei_hbm.at[pl.ds(0, _RB), 0], idx_v, semi).wait()
        pltpu.make_async_copy(ea_hbm.at[0, pl.ds(0, _RB), 0], col_v, semc).wait()

    def compute(idx_v, col_v):
        @plsc.parallel_loop(0, _RB, unroll=2)
        def _(r):
            for j in range(8):
                idx = idx_v[r, pl.ds(j * 16, 16)]
                v = col_v[r, pl.ds(j * 16, 16)]
                plsc.addupdate_scatter(acc, [idx], v)

    for b in range(_NBUF):
        issue(b, idx_vs[b], col_vs[b])

    @pl.loop(0, _NCH, step=_NBUF)
    def _(k):
        for b in range(_NBUF):
            wait(idx_vs[b], col_vs[b])
            compute(idx_vs[b], col_vs[b])

            @pl.when(k + b + _NBUF < _NCH)
            def _():
                issue(k + b + _NBUF, idx_vs[b], col_vs[b])

    @pl.when(c == 0)
    def _():
        pltpu.sync_copy(acc, rect_hbm.at[s])

    @pl.when(c == 1)
    def _():
        pltpu.sync_copy(acc, sent_hbm.at[s])


_seg = functools.partial(
    pl.kernel,
    out_type=[jax.ShapeDtypeStruct((D_EDGE, N_NODES), jnp.float32),
              jax.ShapeDtypeStruct((D_EDGE, N_NODES), jnp.float32)],
    mesh=plsc.VectorSubcoreMesh(core_axis_name="c", subcore_axis_name="s"),
    scratch_types=[pltpu.VMEM((_RB, 128), jnp.int32)] * _NBUF
    + [pltpu.VMEM((_RB, 128), jnp.float32)] * _NBUF
    + [
        pltpu.VMEM((N_NODES,), jnp.float32),
        pltpu.SemaphoreType.DMA,
        pltpu.SemaphoreType.DMA,
    ],
    compiler_params=pltpu.CompilerParams(use_tc_tiling_on_sc=False,
                                         needs_layout_passes=False),
)(_seg_body)


_BM = 2048


def _mm_body(rect_ref, sent_ref, node_ref, g_ref, w_ref, b_ref, out_ref):
    acc = jnp.dot(node_ref[...], w_ref[2 * D_EDGE:2 * D_EDGE + D_NODE, :],
                  preferred_element_type=jnp.float32)
    acc += lax.dot_general(rect_ref[...], w_ref[:D_EDGE, :],
                           (((0,), (0,)), ((), ())),
                           preferred_element_type=jnp.float32)
    acc += lax.dot_general(sent_ref[...], w_ref[D_EDGE:2 * D_EDGE, :],
                           (((0,), (0,)), ((), ())),
                           preferred_element_type=jnp.float32)
    acc += jnp.dot(g_ref[...], w_ref[2 * D_EDGE + D_NODE:, :],
                   preferred_element_type=jnp.float32)
    out_ref[...] = acc + b_ref[...]


def _matmul(rect, sent, node, g2, w, b2):
    d_in = 2 * D_EDGE + D_NODE + D_GLOBAL
    return pl.pallas_call(
        _mm_body,
        grid=(pl.cdiv(N_NODES, _BM),),
        in_specs=[
            pl.BlockSpec((D_EDGE, _BM), lambda i: (0, i)),
            pl.BlockSpec((D_EDGE, _BM), lambda i: (0, i)),
            pl.BlockSpec((_BM, D_NODE), lambda i: (i, 0)),
            pl.BlockSpec((1, D_GLOBAL), lambda i: (0, 0)),
            pl.BlockSpec((d_in, D_NODE), lambda i: (0, 0)),
            pl.BlockSpec((1, D_NODE), lambda i: (0, 0)),
        ],
        out_specs=pl.BlockSpec((_BM, D_NODE), lambda i: (i, 0)),
        out_shape=jax.ShapeDtypeStruct((N_NODES, D_NODE), jnp.float32),
    )(rect, sent, node, g2, w, b2)


def kernel(node_attributes, edge_attributes, global_attributes, edge_index, W, b):
    eiv = edge_index.reshape(2, _NRB, 128).transpose(1, 0, 2)
    eav = (edge_attributes.T.reshape(2, 8, _NRB, 128).transpose(0, 2, 1, 3))
    rect, sent = _seg(eiv, eav)
    return _matmul(rect, sent, node_attributes,
                   global_attributes.reshape(1, D_GLOBAL), W,
                   b.reshape(1, D_NODE))
